# baseline (speedup 1.0000x reference)
import functools

import jax
import jax.numpy as jnp
from jax import lax
from jax.experimental import pallas as pl
from jax.experimental.pallas import tpu as pltpu
from jax.experimental.pallas import tpu_sc as plsc

HIDDEN = 4096
NUM_EXPERTS = 64
TOP_K = 8
BLOCK_T = 512
L = 16


def _matmul_block(x_ref, w_ref, s_ref):
    x = x_ref[...]
    w = w_ref[...]
    logits = jax.lax.dot_general(
        x, w, (((1,), (1,)), ((), ())), preferred_element_type=jnp.float32
    )
    s_ref[...] = jnp.maximum(logits, 0.0)


def _tc_scores(hidden_states, W):
    tokens = hidden_states.shape[0]
    grid = (tokens // BLOCK_T,)
    return pl.pallas_call(
        _matmul_block,
        grid=grid,
        in_specs=[
            pl.BlockSpec((BLOCK_T, HIDDEN), lambda i: (i, 0)),
            pl.BlockSpec((NUM_EXPERTS, HIDDEN), lambda i: (0, 0)),
        ],
        out_specs=pl.BlockSpec((BLOCK_T, NUM_EXPERTS), lambda i: (i, 0)),
        out_shape=jax.ShapeDtypeStruct((tokens, NUM_EXPERTS), jnp.float32),
    )(hidden_states, W)


def _make_sc_topk(tokens):
    info = plsc.get_sparse_core_info()
    nw = info.num_cores * info.num_subcores
    t_per_w = tokens // nw
    mesh = plsc.VectorSubcoreMesh(core_axis_name="c", subcore_axis_name="s")

    @functools.partial(
        pl.kernel,
        mesh=mesh,
        compiler_params=pltpu.CompilerParams(needs_layout_passes=False),
        out_type=[
            jax.ShapeDtypeStruct((tokens * TOP_K,), jnp.float32),
            jax.ShapeDtypeStruct((tokens * TOP_K,), jnp.int32),
        ],
        scratch_types=[
            pltpu.VMEM((t_per_w * NUM_EXPERTS,), jnp.float32),
            pltpu.VMEM((t_per_w * TOP_K,), jnp.float32),
            pltpu.VMEM((t_per_w * TOP_K,), jnp.int32),
        ],
    )
    def sc_topk(s_hbm, tw_hbm, ti_hbm, s_v, tw_v, ti_v):
        wid = lax.axis_index("s") * info.num_cores + lax.axis_index("c")
        base = wid * t_per_w
        pltpu.sync_copy(s_hbm.at[pl.ds(base * NUM_EXPERTS, t_per_w * NUM_EXPERTS)], s_v)

        lane = lax.iota(jnp.int32, L)

        def group(g, carry):
            t0 = g * L
            sidx = t0 * NUM_EXPERTS + lane * NUM_EXPERTS
            oidx = t0 * TOP_K + lane * TOP_K
            vals = []
            idxs = []
            sums = jnp.zeros((L,), jnp.float32)
            for k in range(TOP_K):
                m = jnp.full((L,), -1.0, jnp.float32)
                mi = jnp.zeros((L,), jnp.int32)
                for e in range(NUM_EXPERTS):
                    v = plsc.load_gather(s_v, [sidx + e])
                    c = v > m
                    m = jnp.where(c, v, m)
                    mi = jnp.where(c, jnp.int32(e), mi)
                plsc.store_scatter(s_v, [sidx + mi], jnp.full((L,), -1.0, jnp.float32))
                vals.append(m)
                idxs.append(mi)
                sums = sums + m
            inv = 1.0 / (sums + 1e-6)
            for k in range(TOP_K):
                plsc.store_scatter(tw_v, [oidx + k], vals[k] * inv)
                plsc.store_scatter(ti_v, [oidx + k], idxs[k])
            return carry

        lax.fori_loop(0, t_per_w // L, group, 0)
        pltpu.sync_copy(tw_v, tw_hbm.at[pl.ds(base * TOP_K, t_per_w * TOP_K)])
        pltpu.sync_copy(ti_v, ti_hbm.at[pl.ds(base * TOP_K, t_per_w * TOP_K)])

    return sc_topk


@jax.jit
def kernel(hidden_states, W):
    tokens = hidden_states.shape[0]
    scores = _tc_scores(hidden_states, W)
    tw_f, ti_f = _make_sc_topk(tokens)(scores.reshape(-1))
    return tw_f.reshape(tokens, TOP_K), ti_f.reshape(tokens, TOP_K)


# speedup vs baseline: 2.7144x; 2.7144x over previous
import functools

import jax
import jax.numpy as jnp
from jax.experimental import pallas as pl

HIDDEN = 4096
NUM_EXPERTS = 64
TOP_K = 8
BLOCK_T = 1024


def _router_block(x_ref, w_ref, tw_ref, ti_ref):
    x = x_ref[...]
    w = w_ref[...]
    logits = jax.lax.dot_general(
        x, w, (((1,), (1,)), ((), ())), preferred_element_type=jnp.float32
    )
    scores = jnp.maximum(logits, 0.0)

    bt = scores.shape[0]
    s = scores.T
    lanef = jax.lax.broadcasted_iota(jnp.int32, (NUM_EXPERTS, bt), 0).astype(
        jnp.float32
    )
    vals = []
    idxs = []
    for _ in range(TOP_K):
        m = jnp.max(s, axis=0, keepdims=True)
        i = jnp.min(
            jnp.where(s == m, lanef, float(NUM_EXPERTS)), axis=0, keepdims=True
        )
        vals.append(m)
        idxs.append(i)
        s = jnp.where(lanef == i, -1.0, s)
    tw = jnp.concatenate(vals, axis=0)
    ti = jnp.concatenate(idxs, axis=0).astype(jnp.int32)
    tw = tw / (jnp.sum(tw, axis=0, keepdims=True) + 1e-6)
    tw_ref[...] = tw.T
    ti_ref[...] = ti.T


@jax.jit
def kernel(hidden_states, W):
    tokens = hidden_states.shape[0]
    grid = (tokens // BLOCK_T,)
    tw_t, ti_t = pl.pallas_call(
        _router_block,
        grid=grid,
        in_specs=[
            pl.BlockSpec((BLOCK_T, HIDDEN), lambda i: (i, 0)),
            pl.BlockSpec((NUM_EXPERTS, HIDDEN), lambda i: (0, 0)),
        ],
        out_specs=[
            pl.BlockSpec((BLOCK_T, TOP_K), lambda i: (i, 0)),
            pl.BlockSpec((BLOCK_T, TOP_K), lambda i: (i, 0)),
        ],
        out_shape=[
            jax.ShapeDtypeStruct((tokens, TOP_K), jnp.float32),
            jax.ShapeDtypeStruct((tokens, TOP_K), jnp.int32),
        ],
    )(hidden_states, W)
    return tw_t, ti_t


# speedup vs baseline: 3.2102x; 1.1827x over previous
import functools

import jax
import jax.numpy as jnp
from jax.experimental import pallas as pl

HIDDEN = 4096
NUM_EXPERTS = 64
TOP_K = 8
BLOCK_T = 1024


def _router_block(x_ref, w_ref, tw_ref, ti_ref):
    x = x_ref[...]
    w = w_ref[...]
    logits = jax.lax.dot_general(
        x, w, (((1,), (1,)), ((), ())), preferred_element_type=jnp.float32
    )
    scores = jnp.maximum(logits, 0.0)

    bt = scores.shape[0]
    s = scores.T
    lanef = jax.lax.broadcasted_iota(jnp.int32, (NUM_EXPERTS, bt), 0).astype(
        jnp.float32
    )
    vals = []
    idxs = []
    for _ in range(TOP_K):
        m = jnp.max(s, axis=0, keepdims=True)
        i = jnp.min(
            jnp.where(s == m, lanef, float(NUM_EXPERTS)), axis=0, keepdims=True
        )
        vals.append(m)
        idxs.append(i)
        s = jnp.where(lanef == i, -1.0, s)
    tw = jnp.concatenate(vals, axis=0)
    ti = jnp.concatenate(idxs, axis=0).astype(jnp.int32)
    tw = tw / (jnp.sum(tw, axis=0, keepdims=True) + 1e-6)
    tw_ref[...] = tw
    ti_ref[...] = ti


@jax.jit
def kernel(hidden_states, W):
    tokens = hidden_states.shape[0]
    grid = (tokens // BLOCK_T,)
    tw_t, ti_t = pl.pallas_call(
        _router_block,
        grid=grid,
        in_specs=[
            pl.BlockSpec((BLOCK_T, HIDDEN), lambda i: (i, 0)),
            pl.BlockSpec((NUM_EXPERTS, HIDDEN), lambda i: (0, 0)),
        ],
        out_specs=[
            pl.BlockSpec((TOP_K, BLOCK_T), lambda i: (0, i)),
            pl.BlockSpec((TOP_K, BLOCK_T), lambda i: (0, i)),
        ],
        out_shape=[
            jax.ShapeDtypeStruct((TOP_K, tokens), jnp.float32),
            jax.ShapeDtypeStruct((TOP_K, tokens), jnp.int32),
        ],
    )(hidden_states, W)
    return tw_t.T, ti_t.T


# speedup vs baseline: 3.2107x; 1.0001x over previous
import functools

import jax
import jax.numpy as jnp
from jax.experimental import pallas as pl
from jax.experimental.pallas import tpu as pltpu

HIDDEN = 4096
NUM_EXPERTS = 64
TOP_K = 8
BLOCK_T = 1024


def _router_block(x_ref, w_ref, tw_ref, ti_ref, s_ref):
    x = x_ref[...]
    w = w_ref[...]
    logits = jax.lax.dot_general(
        x, w, (((1,), (1,)), ((), ())), preferred_element_type=jnp.float32
    )
    s_ref[...] = jnp.maximum(logits, 0.0)
    scores = s_ref[...]

    bt = scores.shape[0]
    s = scores.T
    lanef = jax.lax.broadcasted_iota(jnp.int32, (NUM_EXPERTS, bt), 0).astype(
        jnp.float32
    )
    vals = []
    idxs = []
    for _ in range(TOP_K):
        m = jnp.max(s, axis=0, keepdims=True)
        i = jnp.min(
            jnp.where(s == m, lanef, float(NUM_EXPERTS)), axis=0, keepdims=True
        )
        vals.append(m)
        idxs.append(i)
        s = jnp.where(lanef == i, -1.0, s)
    tw = jnp.concatenate(vals, axis=0)
    ti = jnp.concatenate(idxs, axis=0).astype(jnp.int32)
    tw = tw / (jnp.sum(tw, axis=0, keepdims=True) + 1e-6)
    tw_ref[...] = tw
    ti_ref[...] = ti


@jax.jit
def kernel(hidden_states, W):
    tokens = hidden_states.shape[0]
    grid = (tokens // BLOCK_T,)
    tw_t, ti_t = pl.pallas_call(
        _router_block,
        grid=grid,
        in_specs=[
            pl.BlockSpec((BLOCK_T, HIDDEN), lambda i: (i, 0)),
            pl.BlockSpec((NUM_EXPERTS, HIDDEN), lambda i: (0, 0)),
        ],
        out_specs=[
            pl.BlockSpec((TOP_K, BLOCK_T), lambda i: (0, i)),
            pl.BlockSpec((TOP_K, BLOCK_T), lambda i: (0, i)),
        ],
        out_shape=[
            jax.ShapeDtypeStruct((TOP_K, tokens), jnp.float32),
            jax.ShapeDtypeStruct((TOP_K, tokens), jnp.int32),
        ],
        scratch_shapes=[pltpu.VMEM((BLOCK_T, NUM_EXPERTS), jnp.float32)],
    )(hidden_states, W)
    return tw_t.T, ti_t.T


# speedup vs baseline: 3.2116x; 1.0003x over previous
import jax
import jax.numpy as jnp
from jax.experimental import pallas as pl

HIDDEN = 4096
NUM_EXPERTS = 64
TOP_K = 8
BLOCK_T = 1024


def _router_block(x_ref, w_ref, tw_ref, ti_ref):
    x = x_ref[...]
    w = w_ref[...]
    logits = jax.lax.dot_general(
        x, w, (((1,), (1,)), ((), ())), preferred_element_type=jnp.float32
    )
    scores = jnp.maximum(logits, 0.0)

    bt = scores.shape[0]
    s = scores.T
    lanef = jax.lax.broadcasted_iota(jnp.int32, (NUM_EXPERTS, bt), 0).astype(
        jnp.float32
    )
    vals = []
    idxs = []
    for _ in range(TOP_K):
        m = jnp.max(s, axis=0, keepdims=True)
        i = jnp.min(
            jnp.where(s == m, lanef, float(NUM_EXPERTS)), axis=0, keepdims=True
        )
        vals.append(m)
        idxs.append(i)
        s = jnp.where(lanef == i, -1.0, s)
    tw = jnp.concatenate(vals, axis=0)
    ti = jnp.concatenate(idxs, axis=0).astype(jnp.int32)
    tw = tw / (jnp.sum(tw, axis=0, keepdims=True) + 1e-6)
    tw_ref[...] = tw
    ti_ref[...] = ti


@jax.jit
def kernel(hidden_states, W):
    tokens = hidden_states.shape[0]
    grid = (tokens // BLOCK_T,)
    tw_t, ti_t = pl.pallas_call(
        _router_block,
        grid=grid,
        in_specs=[
            pl.BlockSpec((BLOCK_T, HIDDEN), lambda i: (i, 0)),
            pl.BlockSpec((NUM_EXPERTS, HIDDEN), lambda i: (0, 0)),
        ],
        out_specs=[
            pl.BlockSpec((TOP_K, BLOCK_T), lambda i: (0, i)),
            pl.BlockSpec((TOP_K, BLOCK_T), lambda i: (0, i)),
        ],
        out_shape=[
            jax.ShapeDtypeStruct((TOP_K, tokens), jnp.float32),
            jax.ShapeDtypeStruct((TOP_K, tokens), jnp.int32),
        ],
    )(hidden_states, W)
    return tw_t.T, ti_t.T
